# baseline (speedup 1.0000x reference)
import functools

import jax
import jax.numpy as jnp
from jax import lax
from jax.experimental import pallas as pl
from jax.experimental.pallas import tpu as pltpu
from jax.experimental.pallas import tpu_sc as plsc

K = 512
D = 256
NT = 10
NG = 64
N = 3000
NP = 3072
TILE = 256
TEMP = 0.1
SENTINEL = 1000000

_NW = 32
_BPW = NP // _NW


def _vq_body(cb_ref, w1_ref, b1_ref, w2_ref, b2_ref, causal_ref,
             ccb_ref, kcb_ref, cmat_ref, kmat_ref):
    cb = cb_ref[...]
    h = lax.dot_general(cb, w1_ref[...], (((1,), (1,)), ((), ())),
                        preferred_element_type=jnp.float32) + b1_ref[...]
    ct = lax.dot_general(jax.nn.sigmoid(h), w2_ref[...],
                         (((1,), (1,)), ((), ())),
                         preferred_element_type=jnp.float32) + b2_ref[...]
    cs = jax.nn.sigmoid(ct)
    causal = causal_ref[...]
    rows = lax.broadcasted_iota(jnp.int32, (K, K), 0)
    cols = lax.broadcasted_iota(jnp.int32, (K, K), 1)
    eye = rows == cols
    causal_eff = jnp.where(eye, 1.0, causal)
    off = jnp.where(eye, 0.0, causal)
    deg = jnp.sum(jnp.abs(off), axis=1, keepdims=True)
    counter_matrix = off / (deg + 1e-8)
    causal_i = jnp.where(eye, causal, 0.0)
    ccb_ref[...] = jnp.dot(causal_eff,
                           jnp.dot(causal_i, cs,
                                   preferred_element_type=jnp.float32),
                           preferred_element_type=jnp.float32)
    kcb_ref[...] = jnp.dot(causal_eff,
                           jnp.dot(counter_matrix, cs,
                                   preferred_element_type=jnp.float32),
                           preferred_element_type=jnp.float32)
    cmat_ref[...] = causal_i
    kmat_ref[...] = counter_matrix


_vq = pl.pallas_call(
    _vq_body,
    out_shape=[
        jax.ShapeDtypeStruct((K, D), jnp.float32),
        jax.ShapeDtypeStruct((K, D), jnp.float32),
        jax.ShapeDtypeStruct((K, K), jnp.float32),
        jax.ShapeDtypeStruct((K, K), jnp.float32),
    ],
)


def _cdist_body(x_ref, ccb_ref, idx_ref, nx_ref):
    cb = ccb_ref[...]
    xb = x_ref[...]
    c2 = jnp.sum(cb * cb, axis=1, keepdims=True)
    mm = lax.dot_general(cb, xb, (((1,), (1,)), ((), ())),
                         preferred_element_type=jnp.float32)
    d2 = c2 - 2.0 * mm
    m = jnp.min(d2, axis=0, keepdims=True)
    row = lax.broadcasted_iota(jnp.int32, (K, TILE), 0)
    idx_ref[0] = jnp.min(jnp.where(d2 == m, row, K), axis=0, keepdims=True)
    nx_ref[...] = xb / (jnp.sqrt(jnp.sum(xb * xb, axis=1, keepdims=True))
                        + 1e-12)


_cdist = pl.pallas_call(
    _cdist_body,
    grid=(NP // TILE,),
    in_specs=[
        pl.BlockSpec((TILE, D), lambda i: (i, 0)),
        pl.BlockSpec((K, D), lambda i: (0, 0)),
    ],
    out_specs=[
        pl.BlockSpec((1, 1, TILE), lambda i: (i, 0, 0)),
        pl.BlockSpec((TILE, D), lambda i: (i, 0)),
    ],
    out_shape=[
        jax.ShapeDtypeStruct((NP // TILE, 1, TILE), jnp.int32),
        jax.ShapeDtypeStruct((NP, D), jnp.float32),
    ],
)


def _sc_gather_body(idx_hbm, ccb_hbm, kcb_hbm, oc_hbm, ok_hbm,
                    idx_v, rows_c, rows_k, sem_c, sem_k):
    wid = lax.axis_index("s") * 2 + lax.axis_index("c")
    base = wid * _BPW
    pltpu.sync_copy(idx_hbm.at[pl.ds(base, _BPW)], idx_v)
    cpy_c = pltpu.async_copy(ccb_hbm.at[idx_v], rows_c, sem_c)
    cpy_k = pltpu.async_copy(kcb_hbm.at[idx_v], rows_k, sem_k)
    cpy_c.wait()
    cpy_k.wait()
    pltpu.sync_copy(rows_c, oc_hbm.at[pl.ds(base, _BPW)])
    pltpu.sync_copy(rows_k, ok_hbm.at[pl.ds(base, _BPW)])


@functools.lru_cache(maxsize=1)
def _make_sc_gather():
    return pl.kernel(
        _sc_gather_body,
        out_type=[
            jax.ShapeDtypeStruct((NP, D), jnp.float32),
            jax.ShapeDtypeStruct((NP, D), jnp.float32),
        ],
        mesh=plsc.VectorSubcoreMesh(core_axis_name="c",
                                    subcore_axis_name="s"),
        scratch_types=[
            pltpu.VMEM((_BPW,), jnp.int32),
            pltpu.VMEM((_BPW, D), jnp.float32),
            pltpu.VMEM((_BPW, D), jnp.float32),
            pltpu.SemaphoreType.DMA,
            pltpu.SemaphoreType.DMA,
        ],
    )


def _pool_body(x_ref, selc_ref, selk_ref, bf_ref, fcw_ref, fcb_ref,
               prec_ref, prek_ref, prey_ref, cout_ref,
               pc_ref, px_ref, nc_ref):
    bf = bf_ref[...]
    gi = lax.broadcasted_iota(jnp.int32, (NG, NP), 0)
    oh = (jnp.broadcast_to(bf, (NG, NP)) == gi).astype(jnp.float32)
    xp = x_ref[...]
    causal_out = xp + selc_ref[...]
    selk = selk_ref[...]
    cout_ref[...] = causal_out
    cnt = jnp.maximum(jnp.sum(oh, axis=1, keepdims=True), 1.0)
    hp = lax.Precision.HIGHEST
    pooled_x = jnp.dot(oh, xp, preferred_element_type=jnp.float32,
                       precision=hp) / cnt
    pooled_c = jnp.dot(oh, causal_out, preferred_element_type=jnp.float32,
                       precision=hp) / cnt
    pooled_k = jnp.dot(oh, selk, preferred_element_type=jnp.float32,
                       precision=hp) / cnt
    px_ref[...] = pooled_x
    pc_ref[...] = pooled_c
    fcw = fcw_ref[...]
    fcb = fcb_ref[...]
    dn = (((1,), (1,)), ((), ()))
    prec_ref[...] = lax.dot_general(
        pooled_c, fcw, dn, preferred_element_type=jnp.float32) + fcb
    prek_ref[...] = lax.dot_general(
        pooled_k, fcw, dn, preferred_element_type=jnp.float32) + fcb
    prey_ref[...] = lax.dot_general(
        pooled_x, fcw, dn, preferred_element_type=jnp.float32) + fcb
    nc_ref[...] = causal_out / (
        jnp.sqrt(jnp.sum(causal_out * causal_out, axis=1, keepdims=True))
        + 1e-12)


_pool = pl.pallas_call(
    _pool_body,
    out_shape=[
        jax.ShapeDtypeStruct((NG, 128), jnp.float32),
        jax.ShapeDtypeStruct((NG, 128), jnp.float32),
        jax.ShapeDtypeStruct((NG, 128), jnp.float32),
        jax.ShapeDtypeStruct((NP, D), jnp.float32),
        jax.ShapeDtypeStruct((NG, D), jnp.float32),
        jax.ShapeDtypeStruct((NG, D), jnp.float32),
        jax.ShapeDtypeStruct((NP, D), jnp.float32),
    ],
)


ATILE = 512

def _adj_body(sj_ref, nxr_ref, nxc_ref, ncr_ref, ncc_ref, br_ref, bc_ref,
              ao_ref, ar_ref):
    brow = br_ref[...]
    bcol = bc_ref[...]
    cond = jnp.logical_and(jnp.max(brow) >= jnp.min(bcol),
                           jnp.max(bcol) >= jnp.min(brow))

    @pl.when(cond)
    def _():
        mask = (jnp.broadcast_to(brow, (ATILE, ATILE))
                == jnp.broadcast_to(bcol, (ATILE, ATILE)))
        dn = (((1,), (1,)), ((), ()))
        a = lax.dot_general(nxr_ref[...], nxc_ref[...], dn,
                            preferred_element_type=jnp.float32)
        ao_ref[...] = jnp.where(mask, jax.nn.sigmoid(a / TEMP), 0.0)
        b = lax.dot_general(ncr_ref[...], ncc_ref[...], dn,
                            preferred_element_type=jnp.float32)
        ar_ref[...] = jnp.where(mask, jax.nn.sigmoid(b / TEMP), 0.0)

    @pl.when(jnp.logical_not(cond))
    def _():
        zeros = jnp.zeros((ATILE, ATILE), jnp.float32)
        ao_ref[...] = zeros
        ar_ref[...] = zeros


_adj = pl.pallas_call(
    _adj_body,
    grid_spec=pltpu.PrefetchScalarGridSpec(
        num_scalar_prefetch=1,
        grid=(NP // ATILE, NP // ATILE),
        in_specs=[
            pl.BlockSpec((ATILE, D), lambda i, j, sj: (i, 0)),
            pl.BlockSpec((ATILE, D), lambda i, j, sj: (sj[i, j], 0)),
            pl.BlockSpec((ATILE, D), lambda i, j, sj: (i, 0)),
            pl.BlockSpec((ATILE, D), lambda i, j, sj: (sj[i, j], 0)),
            pl.BlockSpec((ATILE, 1), lambda i, j, sj: (i, 0)),
            pl.BlockSpec((1, ATILE), lambda i, j, sj: (0, j)),
        ],
        out_specs=[
            pl.BlockSpec((ATILE, ATILE), lambda i, j, sj: (i, j)),
            pl.BlockSpec((ATILE, ATILE), lambda i, j, sj: (i, j)),
        ],
    ),
    out_shape=[
        jax.ShapeDtypeStruct((N, N), jnp.float32),
        jax.ShapeDtypeStruct((N, N), jnp.float32),
    ],
)


def kernel(x, batch, codebook_input, W1, b1, W2, b2, causal, fc_w, fc_b):
    causal_cb, counter_cb, causal_matrix, counter_matrix = _vq(
        codebook_input, W1, b1.reshape(1, 128), W2, b2.reshape(1, D),
        causal)

    xp = jnp.pad(x, ((0, NP - N), (0, 0)))
    batch_p = jnp.pad(batch.astype(jnp.int32), (0, NP - N),
                      constant_values=SENTINEL)
    idx3, nx = _cdist(xp, causal_cb)
    idx = idx3.reshape(NP)

    selc_p, selk_p = _make_sc_gather()(idx, causal_cb, counter_cb)

    bt = batch_p.reshape(NP // ATILE, ATILE)
    rmin, rmax = bt[:, 0], bt[:, -1]
    active = ((rmax[:, None] >= rmin[None, :])
              & (rmax[None, :] >= rmin[:, None]))
    cols = jnp.arange(NP // ATILE, dtype=jnp.int32)
    jmin = jnp.min(jnp.where(active, cols[None, :], NP // ATILE), axis=1)
    jmax = jnp.max(jnp.where(active, cols[None, :], -1), axis=1)
    sj = jnp.clip(cols[None, :], jmin[:, None], jmax[:, None])
    sj = sj.astype(jnp.int32)

    bf = batch_p.reshape(1, NP)
    brp = batch_p.reshape(NP, 1)
    fcw_p = jnp.pad(fc_w, ((0, 128 - NT), (0, 0)))
    fcb_p = jnp.pad(fc_b, (0, 128 - NT)).reshape(1, 128)
    (pre_c, pre_k, pre_y, causal_out_p, pooled_c, pooled_x,
     nc) = _pool(xp, selc_p, selk_p, bf, fcw_p, fcb_p)

    a_ori, a_rec = _adj(sj, nx, nx, nc, nc, brp, bf)

    return (pre_c[:, :NT], pre_k[:, :NT], pre_y[:, :NT],
            a_ori, a_rec, causal_out_p[:N], x,
            causal_matrix, counter_matrix, pooled_c, pooled_x)


# speedup vs baseline: 1.6544x; 1.6544x over previous
import functools

import jax
import jax.numpy as jnp
from jax import lax
from jax.experimental import pallas as pl
from jax.experimental.pallas import tpu as pltpu
from jax.experimental.pallas import tpu_sc as plsc

K = 512
D = 256
NT = 10
NG = 64
N = 3000
NP = 3072
TILE = 256
TEMP = 0.1
SENTINEL = 1000000

_NW = 32
_BPW = NP // _NW
_REP = 32


def _vq_body(cb_ref, w1_ref, b1_ref, w2_ref, b2_ref, causal_ref,
             ccb_ref, kcb_ref, cmat_ref, kmat_ref):
    cb = cb_ref[...]
    h = lax.dot_general(cb, w1_ref[...], (((1,), (1,)), ((), ())),
                        preferred_element_type=jnp.float32) + b1_ref[...]
    ct = lax.dot_general(jax.nn.sigmoid(h), w2_ref[...],
                         (((1,), (1,)), ((), ())),
                         preferred_element_type=jnp.float32) + b2_ref[...]
    cs = jax.nn.sigmoid(ct)
    causal = causal_ref[...]
    rows = lax.broadcasted_iota(jnp.int32, (K, K), 0)
    cols = lax.broadcasted_iota(jnp.int32, (K, K), 1)
    eye = rows == cols
    causal_eff = jnp.where(eye, 1.0, causal)
    off = jnp.where(eye, 0.0, causal)
    deg = jnp.sum(jnp.abs(off), axis=1, keepdims=True)
    counter_matrix = off / (deg + 1e-8)
    causal_i = jnp.where(eye, causal, 0.0)
    ccb_ref[...] = jnp.dot(causal_eff,
                           jnp.dot(causal_i, cs,
                                   preferred_element_type=jnp.float32),
                           preferred_element_type=jnp.float32)
    kcb_ref[...] = jnp.dot(causal_eff,
                           jnp.dot(counter_matrix, cs,
                                   preferred_element_type=jnp.float32),
                           preferred_element_type=jnp.float32)
    cmat_ref[...] = causal_i
    kmat_ref[...] = counter_matrix


_vq = pl.pallas_call(
    _vq_body,
    out_shape=[
        jax.ShapeDtypeStruct((K, D), jnp.float32),
        jax.ShapeDtypeStruct((K, D), jnp.float32),
        jax.ShapeDtypeStruct((K, K), jnp.float32),
        jax.ShapeDtypeStruct((K, K), jnp.float32),
    ],
)


def _cdist_body(x_ref, ccb_ref, idx_ref, nx_ref):
    cb = ccb_ref[...]
    xb = x_ref[...]
    c2 = jnp.sum(cb * cb, axis=1, keepdims=True)
    mm = lax.dot_general(cb, xb, (((1,), (1,)), ((), ())),
                         preferred_element_type=jnp.float32)
    d2 = c2 - 2.0 * mm
    m = jnp.min(d2, axis=0, keepdims=True)
    row = lax.broadcasted_iota(jnp.int32, (K, TILE), 0)
    idx_ref[0] = jnp.min(jnp.where(d2 == m, row, K), axis=0, keepdims=True)
    nx_ref[...] = xb / (jnp.sqrt(jnp.sum(xb * xb, axis=1, keepdims=True))
                        + 1e-12)


_cdist = pl.pallas_call(
    _cdist_body,
    grid=(NP // TILE,),
    in_specs=[
        pl.BlockSpec((TILE, D), lambda i: (i, 0)),
        pl.BlockSpec((K, D), lambda i: (0, 0)),
    ],
    out_specs=[
        pl.BlockSpec((1, 1, TILE), lambda i: (i, 0, 0)),
        pl.BlockSpec((TILE, D), lambda i: (i, 0)),
    ],
    out_shape=[
        jax.ShapeDtypeStruct((NP // TILE, 1, TILE), jnp.int32),
        jax.ShapeDtypeStruct((NP, D), jnp.float32),
    ],
)


def _sc_gather_body(idx_hbm, ccb_hbm, kcb_hbm, oc_hbm, ok_hbm,
                    idx_v, idx2_v, rows_c, rows_k, sem_c, sem_k):
    wid = lax.axis_index("s") * 2 + lax.axis_index("c")
    base = wid * _BPW
    off = (wid % _REP) * K
    pltpu.sync_copy(idx_hbm.at[pl.ds(base, _BPW)], idx_v)
    for ch in range(_BPW // 16):
        idx2_v[pl.ds(ch * 16, 16)] = idx_v[pl.ds(ch * 16, 16)] + off
    cpy_c = pltpu.async_copy(ccb_hbm.at[idx2_v], rows_c, sem_c)
    cpy_k = pltpu.async_copy(kcb_hbm.at[idx2_v], rows_k, sem_k)
    cpy_c.wait()
    cpy_k.wait()
    pltpu.sync_copy(rows_c, oc_hbm.at[pl.ds(base, _BPW)])
    pltpu.sync_copy(rows_k, ok_hbm.at[pl.ds(base, _BPW)])


@functools.lru_cache(maxsize=1)
def _make_sc_gather():
    return pl.kernel(
        _sc_gather_body,
        out_type=[
            jax.ShapeDtypeStruct((NP, D), jnp.float32),
            jax.ShapeDtypeStruct((NP, D), jnp.float32),
        ],
        mesh=plsc.VectorSubcoreMesh(core_axis_name="c",
                                    subcore_axis_name="s"),
        scratch_types=[
            pltpu.VMEM((_BPW,), jnp.int32),
            pltpu.VMEM((_BPW,), jnp.int32),
            pltpu.VMEM((_BPW, D), jnp.float32),
            pltpu.VMEM((_BPW, D), jnp.float32),
            pltpu.SemaphoreType.DMA,
            pltpu.SemaphoreType.DMA,
        ],
    )


def _pool_body(x_ref, selc_ref, selk_ref, bf_ref, fcw_ref, fcb_ref,
               prec_ref, prek_ref, prey_ref, cout_ref,
               pc_ref, px_ref, nc_ref):
    bf = bf_ref[...]
    gi = lax.broadcasted_iota(jnp.int32, (NG, NP), 0)
    oh = (jnp.broadcast_to(bf, (NG, NP)) == gi).astype(jnp.float32)
    xp = x_ref[...]
    causal_out = xp + selc_ref[...]
    selk = selk_ref[...]
    cout_ref[...] = causal_out
    cnt = jnp.maximum(jnp.sum(oh, axis=1, keepdims=True), 1.0)
    hp = lax.Precision.HIGHEST
    pooled_x = jnp.dot(oh, xp, preferred_element_type=jnp.float32,
                       precision=hp) / cnt
    pooled_c = jnp.dot(oh, causal_out, preferred_element_type=jnp.float32,
                       precision=hp) / cnt
    pooled_k = jnp.dot(oh, selk, preferred_element_type=jnp.float32,
                       precision=hp) / cnt
    px_ref[...] = pooled_x
    pc_ref[...] = pooled_c
    fcw = fcw_ref[...]
    fcb = fcb_ref[...]
    dn = (((1,), (1,)), ((), ()))
    prec_ref[...] = lax.dot_general(
        pooled_c, fcw, dn, preferred_element_type=jnp.float32) + fcb
    prek_ref[...] = lax.dot_general(
        pooled_k, fcw, dn, preferred_element_type=jnp.float32) + fcb
    prey_ref[...] = lax.dot_general(
        pooled_x, fcw, dn, preferred_element_type=jnp.float32) + fcb
    nc_ref[...] = causal_out / (
        jnp.sqrt(jnp.sum(causal_out * causal_out, axis=1, keepdims=True))
        + 1e-12)


_pool = pl.pallas_call(
    _pool_body,
    out_shape=[
        jax.ShapeDtypeStruct((NG, 128), jnp.float32),
        jax.ShapeDtypeStruct((NG, 128), jnp.float32),
        jax.ShapeDtypeStruct((NG, 128), jnp.float32),
        jax.ShapeDtypeStruct((NP, D), jnp.float32),
        jax.ShapeDtypeStruct((NG, D), jnp.float32),
        jax.ShapeDtypeStruct((NG, D), jnp.float32),
        jax.ShapeDtypeStruct((NP, D), jnp.float32),
    ],
)


ATILE = 512

def _adj_body(sj_ref, nxr_ref, nxc_ref, ncr_ref, ncc_ref, br_ref, bc_ref,
              ao_ref, ar_ref):
    brow = br_ref[...]
    bcol = bc_ref[...]
    cond = jnp.logical_and(jnp.max(brow) >= jnp.min(bcol),
                           jnp.max(bcol) >= jnp.min(brow))

    @pl.when(cond)
    def _():
        mask = (jnp.broadcast_to(brow, (ATILE, ATILE))
                == jnp.broadcast_to(bcol, (ATILE, ATILE)))
        dn = (((1,), (1,)), ((), ()))
        a = lax.dot_general(nxr_ref[...], nxc_ref[...], dn,
                            preferred_element_type=jnp.float32)
        ao_ref[...] = jnp.where(mask, jax.nn.sigmoid(a / TEMP), 0.0)
        b = lax.dot_general(ncr_ref[...], ncc_ref[...], dn,
                            preferred_element_type=jnp.float32)
        ar_ref[...] = jnp.where(mask, jax.nn.sigmoid(b / TEMP), 0.0)

    @pl.when(jnp.logical_not(cond))
    def _():
        zeros = jnp.zeros((ATILE, ATILE), jnp.float32)
        ao_ref[...] = zeros
        ar_ref[...] = zeros


_adj = pl.pallas_call(
    _adj_body,
    grid_spec=pltpu.PrefetchScalarGridSpec(
        num_scalar_prefetch=1,
        grid=(NP // ATILE, NP // ATILE),
        in_specs=[
            pl.BlockSpec((ATILE, D), lambda i, j, sj: (i, 0)),
            pl.BlockSpec((ATILE, D), lambda i, j, sj: (sj[i, j], 0)),
            pl.BlockSpec((ATILE, D), lambda i, j, sj: (i, 0)),
            pl.BlockSpec((ATILE, D), lambda i, j, sj: (sj[i, j], 0)),
            pl.BlockSpec((ATILE, 1), lambda i, j, sj: (i, 0)),
            pl.BlockSpec((1, ATILE), lambda i, j, sj: (0, j)),
        ],
        out_specs=[
            pl.BlockSpec((ATILE, ATILE), lambda i, j, sj: (i, j)),
            pl.BlockSpec((ATILE, ATILE), lambda i, j, sj: (i, j)),
        ],
    ),
    out_shape=[
        jax.ShapeDtypeStruct((N, N), jnp.float32),
        jax.ShapeDtypeStruct((N, N), jnp.float32),
    ],
)


def kernel(x, batch, codebook_input, W1, b1, W2, b2, causal, fc_w, fc_b):
    causal_cb, counter_cb, causal_matrix, counter_matrix = _vq(
        codebook_input, W1, b1.reshape(1, 128), W2, b2.reshape(1, D),
        causal)

    xp = jnp.pad(x, ((0, NP - N), (0, 0)))
    batch_p = jnp.pad(batch.astype(jnp.int32), (0, NP - N),
                      constant_values=SENTINEL)
    idx3, nx = _cdist(xp, causal_cb)
    idx = idx3.reshape(NP)

    selc_p, selk_p = _make_sc_gather()(
        idx, jnp.tile(causal_cb, (_REP, 1)), jnp.tile(counter_cb, (_REP, 1)))

    bt = batch_p.reshape(NP // ATILE, ATILE)
    rmin, rmax = bt[:, 0], bt[:, -1]
    active = ((rmax[:, None] >= rmin[None, :])
              & (rmax[None, :] >= rmin[:, None]))
    cols = jnp.arange(NP // ATILE, dtype=jnp.int32)
    jmin = jnp.min(jnp.where(active, cols[None, :], NP // ATILE), axis=1)
    jmax = jnp.max(jnp.where(active, cols[None, :], -1), axis=1)
    sj = jnp.clip(cols[None, :], jmin[:, None], jmax[:, None])
    sj = sj.astype(jnp.int32)

    bf = batch_p.reshape(1, NP)
    brp = batch_p.reshape(NP, 1)
    fcw_p = jnp.pad(fc_w, ((0, 128 - NT), (0, 0)))
    fcb_p = jnp.pad(fc_b, (0, 128 - NT)).reshape(1, 128)
    (pre_c, pre_k, pre_y, causal_out_p, pooled_c, pooled_x,
     nc) = _pool(xp, selc_p, selk_p, bf, fcw_p, fcb_p)

    a_ori, a_rec = _adj(sj, nx, nx, nc, nc, brp, bf)

    return (pre_c[:, :NT], pre_k[:, :NT], pre_y[:, :NT],
            a_ori, a_rec, causal_out_p[:N], x,
            causal_matrix, counter_matrix, pooled_c, pooled_x)
